# Initial kernel scaffold; baseline (speedup 1.0000x reference)
import jax
import jax.numpy as jnp
from jax.experimental import pallas as pl


def kernel(x, edge_index, W1, b1, W2, b2, Wlin, blin):
    raise NotImplementedError("write your pallas kernel here")


# speedup vs baseline: 12.9205x; 12.9205x over previous
import functools

import jax
import jax.numpy as jnp
from jax import lax
from jax.experimental import pallas as pl
from jax.experimental.pallas import tpu as pltpu
from jax.experimental.pallas import tpu_sc as plsc

NC = 2
NS = 16
NW = NC * NS
LANES = 16
CHUNK = 80


def _mesh():
    return plsc.VectorSubcoreMesh(
        core_axis_name="c", subcore_axis_name="s", num_cores=NC, num_subcores=NS
    )


RC = 80


def _make_deg_kernel(N, E):
    EW = E // NW
    T = EW // CHUNK
    NRC = N // RC
    TR = (NRC + NS - 1) // NS

    @functools.partial(
        pl.kernel,
        out_type=jax.ShapeDtypeStruct((NC, N, LANES), jnp.float32),
        mesh=_mesh(),
        scratch_types=[
            pltpu.VMEM((CHUNK,), jnp.int32),
            pltpu.VMEM((CHUNK, LANES), jnp.float32),
            pltpu.VMEM((RC, LANES), jnp.float32),
            pltpu.VMEM_SHARED((N, LANES), jnp.float32),
        ],
    )
    def deg_kernel(dst_hbm, out_hbm, idx_v, ones_v, buf_v, acc_sh):
        c = lax.axis_index("c")
        s = lax.axis_index("s")
        w = s * NC + c

        zeros16 = jnp.zeros((LANES,), jnp.float32)
        ones16 = jnp.ones((LANES,), jnp.float32)

        def fill_zero(i, carry):
            buf_v[i, :] = zeros16
            return carry

        lax.fori_loop(0, RC, fill_zero, 0)

        def fill_ones(i, carry):
            ones_v[i, :] = ones16
            return carry

        lax.fori_loop(0, CHUNK, fill_ones, 0)

        def zero_chunk(t, carry):
            j = t * NS + s

            @pl.when(j < NRC)
            def _():
                pltpu.sync_copy(buf_v, acc_sh.at[pl.ds(j * RC, RC)])

            return carry

        lax.fori_loop(0, TR, zero_chunk, 0)
        plsc.subcore_barrier()

        base = w * EW

        def step(t, carry):
            pltpu.sync_copy(dst_hbm.at[pl.ds(base + t * CHUNK, CHUNK)], idx_v)
            pltpu.sync_copy(ones_v, acc_sh.at[idx_v], add=True)
            return carry

        lax.fori_loop(0, T, step, 0)
        plsc.subcore_barrier()

        def export_chunk(t, carry):
            j = t * NS + s

            @pl.when(j < NRC)
            def _():
                pltpu.sync_copy(acc_sh.at[pl.ds(j * RC, RC)], buf_v)
                pltpu.sync_copy(buf_v, out_hbm.at[c, pl.ds(j * RC, RC)])

            return carry

        lax.fori_loop(0, TR, export_chunk, 0)

    return deg_kernel


def _make_agg_kernel(N, E, D):
    EW = E // NW
    T = EW // CHUNK
    NRC = N // RC
    TR = (NRC + NS - 1) // NS

    @functools.partial(
        pl.kernel,
        out_type=jax.ShapeDtypeStruct((NC, N, D), jnp.float32),
        mesh=_mesh(),
        scratch_types=[
            pltpu.VMEM((CHUNK,), jnp.int32),
            pltpu.VMEM((CHUNK,), jnp.int32),
            pltpu.VMEM((CHUNK, D), jnp.float32),
            pltpu.VMEM((RC, D), jnp.float32),
            pltpu.VMEM_SHARED((N, D), jnp.float32),
            pltpu.SemaphoreType.DMA,
        ],
    )
    def agg_kernel(h_hbm, src_hbm, dst_hbm, out_hbm, src_v, dst_v, rows_v,
                   buf_v, acc_sh, sem):
        c = lax.axis_index("c")
        s = lax.axis_index("s")
        w = s * NC + c

        zeros16 = jnp.zeros((LANES,), jnp.float32)
        groups = D // LANES

        def fill_zero(i, carry):
            r = i // groups
            k = i % groups
            buf_v[r, pl.ds(k * LANES, LANES)] = zeros16
            return carry

        lax.fori_loop(0, RC * groups, fill_zero, 0)

        def zero_chunk(t, carry):
            j = t * NS + s

            @pl.when(j < NRC)
            def _():
                pltpu.sync_copy(buf_v, acc_sh.at[pl.ds(j * RC, RC)])

            return carry

        lax.fori_loop(0, TR, zero_chunk, 0)
        plsc.subcore_barrier()

        base = w * EW

        def step(t, carry):
            pltpu.sync_copy(src_hbm.at[pl.ds(base + t * CHUNK, CHUNK)], src_v)
            pltpu.sync_copy(dst_hbm.at[pl.ds(base + t * CHUNK, CHUNK)], dst_v)
            pltpu.async_copy(h_hbm.at[src_v], rows_v, sem).wait()
            pltpu.sync_copy(rows_v, acc_sh.at[dst_v], add=True)
            return carry

        lax.fori_loop(0, T, step, 0)
        plsc.subcore_barrier()

        def export_chunk(t, carry):
            j = t * NS + s

            @pl.when(j < NRC)
            def _():
                pltpu.sync_copy(acc_sh.at[pl.ds(j * RC, RC)], buf_v)
                pltpu.sync_copy(buf_v, out_hbm.at[c, pl.ds(j * RC, RC)])

            return carry

        lax.fori_loop(0, TR, export_chunk, 0)

    return agg_kernel


def _mm1(x, W, degpart, R=1000):
    N, DIN = x.shape
    DH = W.shape[1]

    def body(x_ref, w_ref, dp_ref, ht_ref, dis_ref):
        dsum = jnp.sum(dp_ref[...], axis=(0, 2), keepdims=True)
        deg = jnp.reshape(dsum, (R, 1)) * (1.0 / LANES) + 1.0
        dis = lax.rsqrt(deg)
        g = jnp.dot(x_ref[...], w_ref[...], preferred_element_type=jnp.float32)
        ht_ref[...] = g * dis
        dis_ref[...] = dis

    return pl.pallas_call(
        body,
        grid=(N // R,),
        in_specs=[
            pl.BlockSpec((R, DIN), lambda i: (i, 0)),
            pl.BlockSpec((DIN, DH), lambda i: (0, 0)),
            pl.BlockSpec((NC, R, LANES), lambda i: (0, i, 0)),
        ],
        out_specs=[
            pl.BlockSpec((R, DH), lambda i: (i, 0)),
            pl.BlockSpec((R, 1), lambda i: (i, 0)),
        ],
        out_shape=[
            jax.ShapeDtypeStruct((N, DH), jnp.float32),
            jax.ShapeDtypeStruct((N, 1), jnp.float32),
        ],
    )(x, W, degpart)


def _mm_mid(spart, ht, dis, b, W, R=1000):
    N, DH = ht.shape
    DO = W.shape[1]

    def body(s_ref, h_ref, d_ref, b_ref, w_ref, o_ref):
        tot = s_ref[0] + s_ref[1] + h_ref[...]
        z = jnp.maximum(tot * d_ref[...] + b_ref[...], 0.0)
        g = jnp.dot(z, w_ref[...], preferred_element_type=jnp.float32)
        o_ref[...] = g * d_ref[...]

    return pl.pallas_call(
        body,
        grid=(N // R,),
        in_specs=[
            pl.BlockSpec((NC, R, DH), lambda i: (0, i, 0)),
            pl.BlockSpec((R, DH), lambda i: (i, 0)),
            pl.BlockSpec((R, 1), lambda i: (i, 0)),
            pl.BlockSpec((DH,), lambda i: (0,)),
            pl.BlockSpec((DH, DO), lambda i: (0, 0)),
        ],
        out_specs=pl.BlockSpec((R, DO), lambda i: (i, 0)),
        out_shape=jax.ShapeDtypeStruct((N, DO), jnp.float32),
    )(spart, ht, dis, b, W)


def _mm_head(spart, ht, dis, b, Wlin, blin, R=1000):
    N, DH = ht.shape
    DO = Wlin.shape[1]

    def body(s_ref, h_ref, d_ref, b_ref, w_ref, bl_ref, o_ref):
        tot = s_ref[0] + s_ref[1] + h_ref[...]
        z = jnp.maximum(tot * d_ref[...] + b_ref[...], 0.0)
        g = jnp.dot(z, w_ref[...], preferred_element_type=jnp.float32)
        o_ref[...] = g + bl_ref[...]

    return pl.pallas_call(
        body,
        grid=(N // R,),
        in_specs=[
            pl.BlockSpec((NC, R, DH), lambda i: (0, i, 0)),
            pl.BlockSpec((R, DH), lambda i: (i, 0)),
            pl.BlockSpec((R, 1), lambda i: (i, 0)),
            pl.BlockSpec((DH,), lambda i: (0,)),
            pl.BlockSpec((DH, DO), lambda i: (0, 0)),
            pl.BlockSpec((DO,), lambda i: (0,)),
        ],
        out_specs=pl.BlockSpec((R, DO), lambda i: (i, 0)),
        out_shape=jax.ShapeDtypeStruct((N, DO), jnp.float32),
    )(spart, ht, dis, b, Wlin, blin)


def kernel(x, edge_index, W1, b1, W2, b2, Wlin, blin):
    N, _ = x.shape
    E = edge_index.shape[1]
    DH = W1.shape[1]

    src = edge_index[0]
    dst = edge_index[1]

    deg_k = _make_deg_kernel(N, E)
    agg_k = _make_agg_kernel(N, E, DH)

    degpart = deg_k(dst)
    ht1, dis = _mm1(x, W1, degpart)
    s1 = agg_k(ht1, src, dst)
    ht2 = _mm_mid(s1, ht1, dis, b1, W2)
    s2 = agg_k(ht2, src, dst)
    return _mm_head(s2, ht2, dis, b2, Wlin, blin)


# speedup vs baseline: 25.1700x; 1.9481x over previous
import functools

import jax
import jax.numpy as jnp
from jax import lax
from jax.experimental import pallas as pl
from jax.experimental.pallas import tpu as pltpu
from jax.experimental.pallas import tpu_sc as plsc

NC = 2
NS = 16
NW = NC * NS
LANES = 16
CHUNK = 40
RC = 40
NB = 5
SB = 50


def _mesh():
    return plsc.VectorSubcoreMesh(
        core_axis_name="c", subcore_axis_name="s", num_cores=NC, num_subcores=NS
    )


def _make_deg_kernel(N, E):
    EW = E // NW
    T = EW // CHUNK
    G = T // NB
    NRC = N // RC
    TR = (NRC + NS - 1) // NS

    @functools.partial(
        pl.kernel,
        out_type=jax.ShapeDtypeStruct((NC, N, LANES), jnp.float32),
        mesh=_mesh(),
        scratch_types=[
            pltpu.VMEM((T, CHUNK), jnp.int32),
            pltpu.VMEM((CHUNK, LANES), jnp.float32),
            pltpu.VMEM((RC, LANES), jnp.float32),
            pltpu.VMEM_SHARED((N, LANES), jnp.float32),
            pltpu.SemaphoreType.DMA((NB,)),
        ],
    )
    def deg_kernel(dst_hbm, out_hbm, idx_v, ones_v, buf_v, acc_sh, ssem):
        c = lax.axis_index("c")
        s = lax.axis_index("s")
        w = s * NC + c

        zeros16 = jnp.zeros((LANES,), jnp.float32)
        ones16 = jnp.ones((LANES,), jnp.float32)

        def fill_zero(i, carry):
            buf_v[i, :] = zeros16
            return carry

        lax.fori_loop(0, RC, fill_zero, 0)

        def fill_ones(i, carry):
            ones_v[i, :] = ones16
            return carry

        lax.fori_loop(0, CHUNK, fill_ones, 0)

        def zero_chunk(t, carry):
            j = t * NS + s

            @pl.when(j < NRC)
            def _():
                pltpu.sync_copy(buf_v, acc_sh.at[pl.ds(j * RC, RC)])

            return carry

        lax.fori_loop(0, TR, zero_chunk, 0)
        plsc.subcore_barrier()

        pltpu.sync_copy(dst_hbm.at[w], idx_v)

        def step(q, carry):
            for b in range(NB):
                t = q * NB + b
                pltpu.make_async_copy(
                    ones_v, acc_sh.at[idx_v.at[t]], ssem.at[b]
                ).start(add=True)
            for b in range(NB):
                t = q * NB + b
                pltpu.make_async_copy(
                    ones_v, acc_sh.at[idx_v.at[t]], ssem.at[b]
                ).wait()
            return carry

        lax.fori_loop(0, G, step, 0)
        plsc.subcore_barrier()

        def export_chunk(t, carry):
            j = t * NS + s

            @pl.when(j < NRC)
            def _():
                pltpu.sync_copy(acc_sh.at[pl.ds(j * RC, RC)], buf_v)
                pltpu.sync_copy(buf_v, out_hbm.at[c, pl.ds(j * RC, RC)])

            return carry

        lax.fori_loop(0, TR, export_chunk, 0)

    return deg_kernel


def _make_agg_kernel(N, E, D):
    DC = D // NC
    EW = E // NS
    T = EW // CHUNK
    G = T // NB
    HALF = G // 2
    NBLK = T // SB
    GPB = SB // NB
    NRC = N // RC
    TR = (NRC + NS - 1) // NS

    @functools.partial(
        pl.kernel,
        out_type=jax.ShapeDtypeStruct((NC, N, DC), jnp.float32),
        mesh=_mesh(),
        scratch_types=[
            pltpu.VMEM((2 * SB * CHUNK,), jnp.int32),
            pltpu.VMEM((T, CHUNK), jnp.int32),
            pltpu.VMEM((2 * NB, CHUNK, DC), jnp.float32),
            pltpu.VMEM_SHARED((N, DC), jnp.float32),
            pltpu.SemaphoreType.DMA((2 * NB,)),
            pltpu.SemaphoreType.DMA((2 * NB,)),
            pltpu.SemaphoreType.DMA,
        ],
        compiler_params=pltpu.CompilerParams(use_tc_tiling_on_sc=False),
    )
    def agg_kernel(h_hbm, src_hbm, dst_hbm, out_hbm, src_v, dst_v, rows_v,
                   acc_sh, gsem, ssem, isem):
        c = lax.axis_index("c")
        s = lax.axis_index("s")

        zeros16 = jnp.zeros((LANES,), jnp.float32)
        groups = DC // LANES
        zbuf = rows_v.at[0]

        def fill_zero(i, carry):
            r = i // groups
            k = i % groups
            zbuf[r, pl.ds(k * LANES, LANES)] = zeros16
            return carry

        lax.fori_loop(0, RC * groups, fill_zero, 0)

        def zero_chunk(t, carry):
            j = t * NS + s

            @pl.when(j < NRC)
            def _():
                pltpu.sync_copy(zbuf, acc_sh.at[pl.ds(j * RC, RC)])

            return carry

        lax.fori_loop(0, TR, zero_chunk, 0)
        plsc.subcore_barrier()

        htab = h_hbm.at[c]
        sbase = s * (T * CHUNK)

        def idx_load_descs(blk):
            bank = lax.rem(blk, 2)
            return (
                pltpu.make_async_copy(
                    src_hbm.at[pl.ds(sbase + blk * (SB * CHUNK), SB * CHUNK)],
                    src_v.at[pl.ds(bank * (SB * CHUNK), SB * CHUNK)], isem),
            )

        pltpu.sync_copy(dst_hbm.at[s], dst_v)
        pltpu.sync_copy(src_hbm.at[pl.ds(sbase, SB * CHUNK)],
                        src_v.at[pl.ds(0, SB * CHUNK)])
        for d in idx_load_descs(1):
            d.start()

        def src_row(t):
            off = lax.rem(t // SB, 2) * (SB * CHUNK) + lax.rem(t, SB) * CHUNK
            return src_v.at[pl.ds(off, CHUNK)]

        def dst_row(t):
            return dst_v.at[t]

        def g_desc(t, slot):
            return pltpu.make_async_copy(
                htab.at[src_row(t)], rows_v.at[slot], gsem.at[slot])

        def s_desc(t, slot):
            return pltpu.make_async_copy(
                rows_v.at[slot], acc_sh.at[dst_row(t)], ssem.at[slot])

        for b in range(NB):
            g_desc(b, b).start()

        def emit_group(g, cur, oth, first_pred, last_ok,
                       idx_wait_pred, idx_start_pred):
            for b in range(NB):
                t = g * NB + b
                g_desc(t, cur + b).wait()
                s_desc(t, cur + b).start(add=True)

            if idx_start_pred is not None:
                @pl.when(idx_start_pred)
                def _():
                    for d in idx_load_descs(g // GPB + 1):
                        d.start()

            if idx_wait_pred is not None:
                @pl.when(idx_wait_pred)
                def _():
                    for d in idx_load_descs(g // GPB + 1):
                        d.wait()

            for b in range(NB):
                if first_pred is None:
                    s_desc((g - 1) * NB + b, oth + b).wait()
                else:
                    @pl.when(first_pred)
                    def _():
                        s_desc((g - 1) * NB + b, oth + b).wait()
                if last_ok is None:
                    g_desc((g + 1) * NB + b, oth + b).start()
                else:
                    @pl.when(last_ok)
                    def _():
                        g_desc((g + 1) * NB + b, oth + b).start()

        def pipe(q, carry):
            emit_group(2 * q, 0, NB, first_pred=q > 0, last_ok=None,
                       idx_wait_pred=None, idx_start_pred=None)
            qm5 = lax.rem(q, 5)
            emit_group(2 * q + 1, NB, 0, first_pred=None,
                       last_ok=q < HALF - 1,
                       idx_wait_pred=jnp.logical_and(qm5 == 4,
                                                     q < HALF - 1),
                       idx_start_pred=jnp.logical_and(
                           jnp.logical_and(qm5 == 0, q >= 5),
                           q <= (NBLK - 2) * GPB // 2))
            return carry

        lax.fori_loop(0, HALF, pipe, 0)

        for b in range(NB):
            s_desc((G - 1) * NB + b, NB + b).wait()

        plsc.subcore_barrier()

        def export_chunk(t, carry):
            j = t * NS + s

            @pl.when(j < NRC)
            def _():
                pltpu.sync_copy(acc_sh.at[pl.ds(j * RC, RC)], zbuf)
                pltpu.sync_copy(zbuf, out_hbm.at[c, pl.ds(j * RC, RC)])

            return carry

        lax.fori_loop(0, TR, export_chunk, 0)

    return agg_kernel


def _mm1(x, W, degpart, R=1000):
    N, DIN = x.shape
    DH = W.shape[1]
    DC = DH // NC

    def body(x_ref, w_ref, dp_ref, ht_ref, dis_ref):
        dsum = jnp.sum(dp_ref[...], axis=(0, 2), keepdims=True)
        deg = jnp.reshape(dsum, (R, 1)) * (1.0 / LANES) + 1.0
        dis = lax.rsqrt(deg)
        g = jnp.dot(x_ref[...], w_ref[...], preferred_element_type=jnp.float32)
        g = g * dis
        ht_ref[0] = g[:, :DC]
        ht_ref[1] = g[:, DC:]
        dis_ref[...] = dis

    return pl.pallas_call(
        body,
        grid=(N // R,),
        in_specs=[
            pl.BlockSpec((R, DIN), lambda i: (i, 0)),
            pl.BlockSpec((DIN, DH), lambda i: (0, 0)),
            pl.BlockSpec((NC, R, LANES), lambda i: (0, i, 0)),
        ],
        out_specs=[
            pl.BlockSpec((NC, R, DC), lambda i: (0, i, 0)),
            pl.BlockSpec((R, 1), lambda i: (i, 0)),
        ],
        out_shape=[
            jax.ShapeDtypeStruct((NC, N, DC), jnp.float32),
            jax.ShapeDtypeStruct((N, 1), jnp.float32),
        ],
    )(x, W, degpart)


def _mm_mid(spart, ht, dis, b, W, R=1000):
    _, N, DC = ht.shape
    DH = W.shape[0]
    DO = W.shape[1]
    DOC = DO // NC

    def body(s_ref, h_ref, d_ref, b_ref, w_ref, o_ref):
        tot = jnp.concatenate(
            [s_ref[0] + h_ref[0], s_ref[1] + h_ref[1]], axis=1)
        z = jnp.maximum(tot * d_ref[...] + b_ref[...], 0.0)
        g = jnp.dot(z, w_ref[...], preferred_element_type=jnp.float32)
        g = g * d_ref[...]
        o_ref[0] = g[:, :DOC]
        o_ref[1] = g[:, DOC:]

    return pl.pallas_call(
        body,
        grid=(N // R,),
        in_specs=[
            pl.BlockSpec((NC, R, DC), lambda i: (0, i, 0)),
            pl.BlockSpec((NC, R, DC), lambda i: (0, i, 0)),
            pl.BlockSpec((R, 1), lambda i: (i, 0)),
            pl.BlockSpec((DH,), lambda i: (0,)),
            pl.BlockSpec((DH, DO), lambda i: (0, 0)),
        ],
        out_specs=pl.BlockSpec((NC, R, DOC), lambda i: (0, i, 0)),
        out_shape=jax.ShapeDtypeStruct((NC, N, DOC), jnp.float32),
    )(spart, ht, dis, b, W)


def _mm_head(spart, ht, dis, b, Wlin, blin, R=1000):
    _, N, DC = ht.shape
    DH = Wlin.shape[0]
    DO = Wlin.shape[1]

    def body(s_ref, h_ref, d_ref, b_ref, w_ref, bl_ref, o_ref):
        tot = jnp.concatenate(
            [s_ref[0] + h_ref[0], s_ref[1] + h_ref[1]], axis=1)
        z = jnp.maximum(tot * d_ref[...] + b_ref[...], 0.0)
        g = jnp.dot(z, w_ref[...], preferred_element_type=jnp.float32)
        o_ref[...] = g + bl_ref[...]

    return pl.pallas_call(
        body,
        grid=(N // R,),
        in_specs=[
            pl.BlockSpec((NC, R, DC), lambda i: (0, i, 0)),
            pl.BlockSpec((NC, R, DC), lambda i: (0, i, 0)),
            pl.BlockSpec((R, 1), lambda i: (i, 0)),
            pl.BlockSpec((DH,), lambda i: (0,)),
            pl.BlockSpec((DH, DO), lambda i: (0, 0)),
            pl.BlockSpec((DO,), lambda i: (0,)),
        ],
        out_specs=pl.BlockSpec((R, DO), lambda i: (i, 0)),
        out_shape=jax.ShapeDtypeStruct((N, DO), jnp.float32),
    )(spart, ht, dis, b, Wlin, blin)


def kernel(x, edge_index, W1, b1, W2, b2, Wlin, blin):
    N, _ = x.shape
    E = edge_index.shape[1]
    DH = W1.shape[1]

    Td = E // NW // CHUNK
    dst_w = edge_index[1].reshape(NW, Td, CHUNK)
    Ta = E // NS // CHUNK
    src_s = edge_index[0]
    dst_s = edge_index[1].reshape(NS, Ta, CHUNK)

    deg_k = _make_deg_kernel(N, E)
    agg_k = _make_agg_kernel(N, E, DH)

    degpart = deg_k(dst_w)
    ht1, dis = _mm1(x, W1, degpart)
    s1 = agg_k(ht1, src_s, dst_s)
    ht2 = _mm_mid(s1, ht1, dis, b1, W2)
    s2 = agg_k(ht2, src_s, dst_s)
    return _mm_head(s2, ht2, dis, b2, Wlin, blin)
